# baseline (speedup 1.0000x reference)
import jax
import jax.numpy as jnp
from jax import lax
from jax.experimental import pallas as pl
from jax.experimental.pallas import tpu as pltpu
from jax.experimental.pallas import tpu_sc as plsc

HIDDEN = 768
NUM_EXPERTS = 8
TOP_K = 2
D_FF = HIDDEN * 2
BM = 128
CH = 512
T_TOK = 2048
A_TOT = T_TOK * TOP_K
P_PAD = A_TOT + NUM_EXPERTS * BM
NTILES = P_PAD // BM

NC = 2
NS = 16
NW = NC * NS
APW = A_TOT // NW
TPW = T_TOK // NW
LANES = 16
WLANES = 32


def _meta_kernel(probs_ref, pos_ref, wexp_ref, te_ref, tv_ref, rank_ref):
    T, E, e_i32 = T_TOK, NUM_EXPERTS, jnp.int32

    probs = probs_ref[...]
    iota = lax.broadcasted_iota(e_i32, (T, E), 1)
    m1 = jnp.max(probs, axis=1, keepdims=True)
    e1 = jnp.min(jnp.where(probs == m1, iota, E), axis=1, keepdims=True)
    probs2 = jnp.where(iota == e1, -1.0, probs)
    m2 = jnp.max(probs2, axis=1, keepdims=True)
    e2 = jnp.min(jnp.where(probs2 == m2, iota, E), axis=1, keepdims=True)
    s = m1 + m2
    wexp_ref[0:T, :] = jnp.broadcast_to(m1 / s, (T, LANES))
    wexp_ref[T:2 * T, :] = jnp.broadcast_to(m2 / s, (T, LANES))

    oh1 = (iota == e1).astype(jnp.float32)
    oh2 = (iota == e2).astype(jnp.float32)

    tri = (lax.broadcasted_iota(e_i32, (CH, CH), 0)
           > lax.broadcasted_iota(e_i32, (CH, CH), 1)).astype(jnp.bfloat16)
    base = jnp.zeros((1, E), jnp.float32)
    for c in range(A_TOT // CH):
        src = oh1 if c < T // CH else oh2
        ohf = src[(c % (T // CH)) * CH:((c % (T // CH)) + 1) * CH, :]
        partial = jnp.dot(tri, ohf.astype(jnp.bfloat16),
                          preferred_element_type=jnp.float32)
        rank_c = partial + base
        rank_ref[c * CH:(c + 1) * CH, :] = (
            jnp.sum(rank_c * ohf, axis=1, keepdims=True))
        base = base + jnp.sum(ohf, axis=0, keepdims=True)

    off = jnp.int32(0)
    starts, ends, offs = [], [], []
    for e in range(E):
        c_e = base[0, e].astype(e_i32)
        pc = ((c_e + BM - 1) // BM) * BM
        offs.append(off)
        starts.append(off // BM)
        ends.append((off + pc) // BM)
        off = off + pc
    prev = jnp.int32(0)
    for t in range(NTILES):
        te_t = jnp.int32(0)
        tv_t = jnp.int32(0)
        for e in range(E):
            inside = ((t >= starts[e]) & (t < ends[e])).astype(e_i32)
            te_t = te_t + inside * e
            tv_t = tv_t | inside
        te_t = jnp.where(tv_t > 0, te_t, prev)
        prev = te_t
        te_ref[t] = te_t
        tv_ref[t] = tv_t

    iota8 = lax.broadcasted_iota(e_i32, (1, E), 1)
    offv = jnp.zeros((1, E), jnp.float32)
    for e in range(E):
        offv = jnp.where(iota8 == e, offs[e].astype(jnp.float32), offv)
    off1 = jnp.sum(oh1 * offv, axis=1, keepdims=True)
    off2 = jnp.sum(oh2 * offv, axis=1, keepdims=True)
    pos_ref[0:T, :] = (rank_ref[0:T, :] + off1).astype(e_i32)
    pos_ref[T:2 * T, :] = (rank_ref[T:2 * T, :] + off2).astype(e_i32)


def _meta(probs):
    return pl.pallas_call(
        _meta_kernel,
        out_shape=[
            jax.ShapeDtypeStruct((A_TOT, 1), jnp.int32),
            jax.ShapeDtypeStruct((A_TOT, LANES), jnp.float32),
            jax.ShapeDtypeStruct((NTILES,), jnp.int32),
            jax.ShapeDtypeStruct((NTILES,), jnp.int32),
        ],
        out_specs=[
            pl.BlockSpec(memory_space=pltpu.VMEM),
            pl.BlockSpec(memory_space=pltpu.VMEM),
            pl.BlockSpec(memory_space=pltpu.SMEM),
            pl.BlockSpec(memory_space=pltpu.SMEM),
        ],
        scratch_shapes=[pltpu.VMEM((A_TOT, 1), jnp.float32)],
    )(probs)


def _dispatch_body(xf_hbm, pos_hbm, xs_hbm, idx_v, rows_v, sem):
    wid = lax.axis_index("s") * NC + lax.axis_index("c")
    base = wid * APW
    srow = lax.rem(base, T_TOK)
    pltpu.sync_copy(pos_hbm.at[pl.ds(base, APW)], idx_v)
    pltpu.sync_copy(xf_hbm.at[pl.ds(srow, APW), :], rows_v)
    pltpu.async_copy(rows_v, xs_hbm.at[idx_v], sem).wait()


def _combine_body(eo_hbm, pos_hbm, wexp_hbm, out_hbm, idx0_v, idx1_v, buf0,
                  buf1, w0_v, w1_v, sem0, sem1):
    wid = lax.axis_index("s") * NC + lax.axis_index("c")
    tb = wid * TPW
    pltpu.sync_copy(pos_hbm.at[pl.ds(tb, TPW)], idx0_v)
    pltpu.sync_copy(pos_hbm.at[pl.ds(T_TOK + tb, TPW)], idx1_v)
    cp0 = pltpu.async_copy(eo_hbm.at[idx0_v], buf0, sem0)
    cp1 = pltpu.async_copy(eo_hbm.at[idx1_v], buf1, sem1)
    pltpu.sync_copy(wexp_hbm.at[pl.ds(tb, TPW), :], w0_v)
    pltpu.sync_copy(wexp_hbm.at[pl.ds(T_TOK + tb, TPW), :], w1_v)
    cp0.wait()
    cp1.wait()

    def body(j, carry):
        w0 = w0_v[j, :]
        w1 = w1_v[j, :]
        for v in range(HIDDEN // LANES):
            sl = pl.ds(v * LANES, LANES)
            buf0[j, sl] = buf0[j, sl] * w0 + buf1[j, sl] * w1
        return carry

    lax.fori_loop(0, TPW, body, 0)
    pltpu.sync_copy(buf0, out_hbm.at[pl.ds(tb, TPW), :])


def _ffn_kernel(te_ref, tv_ref, xs_ref, w1_ref, b1_ref, w2_ref, b2_ref,
                out_ref, w1b, w2b):
    i = pl.program_id(0)
    switched = jnp.logical_or(
        i == 0, te_ref[i] != te_ref[jnp.maximum(i - 1, 0)])

    @pl.when(switched)
    def _():
        w1b[...] = w1_ref[0].astype(jnp.bfloat16)
        w2b[...] = w2_ref[0].astype(jnp.bfloat16)

    @pl.when(tv_ref[i] > 0)
    def _():
        x_t = xs_ref[...].astype(jnp.bfloat16)
        h = jnp.dot(x_t, w1b[...], preferred_element_type=jnp.float32)
        h = jnp.maximum(h + b1_ref[0, 0, :][None, :], 0.0)
        o = jnp.dot(h.astype(jnp.bfloat16), w2b[...],
                    preferred_element_type=jnp.float32)
        out_ref[...] = o + b2_ref[0, 0, :][None, :]


def _grouped_ffn(xs, w1, b1, w2, b2, tile_expert, tile_valid):
    grid_spec = pltpu.PrefetchScalarGridSpec(
        num_scalar_prefetch=2,
        grid=(NTILES,),
        in_specs=[
            pl.BlockSpec((BM, HIDDEN), lambda i, te, tv: (i, 0)),
            pl.BlockSpec((1, HIDDEN, D_FF), lambda i, te, tv: (te[i], 0, 0)),
            pl.BlockSpec((1, 1, D_FF), lambda i, te, tv: (te[i], 0, 0)),
            pl.BlockSpec((1, D_FF, HIDDEN), lambda i, te, tv: (te[i], 0, 0)),
            pl.BlockSpec((1, 1, HIDDEN), lambda i, te, tv: (te[i], 0, 0)),
        ],
        out_specs=pl.BlockSpec((BM, HIDDEN), lambda i, te, tv: (i, 0)),
        scratch_shapes=[
            pltpu.VMEM((HIDDEN, D_FF), jnp.bfloat16),
            pltpu.VMEM((D_FF, HIDDEN), jnp.bfloat16),
        ],
    )
    return pl.pallas_call(
        _ffn_kernel,
        grid_spec=grid_spec,
        out_shape=jax.ShapeDtypeStruct((P_PAD, HIDDEN), jnp.float32),
    )(tile_expert, tile_valid, xs, w1, b1[:, None, :], w2, b2[:, None, :])


@jax.jit
def kernel(x, router_w, router_b, w1, b1, w2, b2):
    B, S, H = x.shape
    T = B * S
    xf = x.reshape(T, H)

    logits = xf @ router_w + router_b
    probs = jax.nn.softmax(logits, axis=-1)

    pos, wexp, te, tv = _meta(probs)
    posf = pos.reshape(A_TOT)

    mesh = plsc.VectorSubcoreMesh(core_axis_name="c", subcore_axis_name="s")
    dispatch = pl.kernel(
        _dispatch_body,
        mesh=mesh,
        out_type=jax.ShapeDtypeStruct((P_PAD, HIDDEN), jnp.float32),
        scratch_types=[
            pltpu.VMEM((APW,), jnp.int32),
            pltpu.VMEM((APW, HIDDEN), jnp.float32),
            pltpu.SemaphoreType.DMA,
        ],
    )
    combine = pl.kernel(
        _combine_body,
        mesh=mesh,
        out_type=jax.ShapeDtypeStruct((T_TOK, HIDDEN), jnp.float32),
        scratch_types=[
            pltpu.VMEM((TPW,), jnp.int32),
            pltpu.VMEM((TPW,), jnp.int32),
            pltpu.VMEM((TPW, HIDDEN), jnp.float32),
            pltpu.VMEM((TPW, HIDDEN), jnp.float32),
            pltpu.VMEM((TPW, LANES), jnp.float32),
            pltpu.VMEM((TPW, LANES), jnp.float32),
            pltpu.SemaphoreType.DMA,
            pltpu.SemaphoreType.DMA,
        ],
    )

    xs = dispatch(xf, posf)
    eo = _grouped_ffn(xs, w1, b1, w2, b2, te, tv)
    out = combine(eo, posf, wexp)
    return out.reshape(B, S, H)


# speedup vs baseline: 1.0674x; 1.0674x over previous
import jax
import jax.numpy as jnp
from jax import lax
from jax.experimental import pallas as pl
from jax.experimental.pallas import tpu as pltpu
from jax.experimental.pallas import tpu_sc as plsc

HIDDEN = 768
NUM_EXPERTS = 8
TOP_K = 2
D_FF = HIDDEN * 2
BM = 256
CH = 512
T_TOK = 2048
A_TOT = T_TOK * TOP_K
P_PAD = A_TOT + NUM_EXPERTS * BM
NTILES = P_PAD // BM

NC = 2
NS = 16
NW = NC * NS
APW = A_TOT // NW
TPW = T_TOK // NW
LANES = 16
WLANES = 32


def _meta_kernel(probs_ref, pos_ref, wexp_ref, te_ref, tv_ref, rank_ref):
    T, E, e_i32 = T_TOK, NUM_EXPERTS, jnp.int32

    probs = probs_ref[...]
    iota = lax.broadcasted_iota(e_i32, (T, E), 1)
    m1 = jnp.max(probs, axis=1, keepdims=True)
    e1 = jnp.min(jnp.where(probs == m1, iota, E), axis=1, keepdims=True)
    probs2 = jnp.where(iota == e1, -1.0, probs)
    m2 = jnp.max(probs2, axis=1, keepdims=True)
    e2 = jnp.min(jnp.where(probs2 == m2, iota, E), axis=1, keepdims=True)
    s = m1 + m2
    wexp_ref[0:T, :] = jnp.broadcast_to(m1 / s, (T, LANES))
    wexp_ref[T:2 * T, :] = jnp.broadcast_to(m2 / s, (T, LANES))

    oh1 = (iota == e1).astype(jnp.float32)
    oh2 = (iota == e2).astype(jnp.float32)

    tri = (lax.broadcasted_iota(e_i32, (CH, CH), 0)
           > lax.broadcasted_iota(e_i32, (CH, CH), 1)).astype(jnp.bfloat16)
    base = jnp.zeros((1, E), jnp.float32)
    for c in range(A_TOT // CH):
        src = oh1 if c < T // CH else oh2
        ohf = src[(c % (T // CH)) * CH:((c % (T // CH)) + 1) * CH, :]
        partial = jnp.dot(tri, ohf.astype(jnp.bfloat16),
                          preferred_element_type=jnp.float32)
        rank_c = partial + base
        rank_ref[c * CH:(c + 1) * CH, :] = (
            jnp.sum(rank_c * ohf, axis=1, keepdims=True))
        base = base + jnp.sum(ohf, axis=0, keepdims=True)

    off = jnp.int32(0)
    starts, ends, offs = [], [], []
    for e in range(E):
        c_e = base[0, e].astype(e_i32)
        pc = ((c_e + BM - 1) // BM) * BM
        offs.append(off)
        starts.append(off // BM)
        ends.append((off + pc) // BM)
        off = off + pc
    prev = jnp.int32(0)
    for t in range(NTILES):
        te_t = jnp.int32(0)
        tv_t = jnp.int32(0)
        for e in range(E):
            inside = ((t >= starts[e]) & (t < ends[e])).astype(e_i32)
            te_t = te_t + inside * e
            tv_t = tv_t | inside
        te_t = jnp.where(tv_t > 0, te_t, prev)
        prev = te_t
        te_ref[t] = te_t
        tv_ref[t] = tv_t

    iota8 = lax.broadcasted_iota(e_i32, (1, E), 1)
    offv = jnp.zeros((1, E), jnp.float32)
    for e in range(E):
        offv = jnp.where(iota8 == e, offs[e].astype(jnp.float32), offv)
    off1 = jnp.sum(oh1 * offv, axis=1, keepdims=True)
    off2 = jnp.sum(oh2 * offv, axis=1, keepdims=True)
    pos_ref[0:T, :] = (rank_ref[0:T, :] + off1).astype(e_i32)
    pos_ref[T:2 * T, :] = (rank_ref[T:2 * T, :] + off2).astype(e_i32)


def _meta(probs):
    return pl.pallas_call(
        _meta_kernel,
        out_shape=[
            jax.ShapeDtypeStruct((A_TOT, 1), jnp.int32),
            jax.ShapeDtypeStruct((A_TOT, LANES), jnp.float32),
            jax.ShapeDtypeStruct((NTILES,), jnp.int32),
            jax.ShapeDtypeStruct((NTILES,), jnp.int32),
        ],
        out_specs=[
            pl.BlockSpec(memory_space=pltpu.VMEM),
            pl.BlockSpec(memory_space=pltpu.VMEM),
            pl.BlockSpec(memory_space=pltpu.SMEM),
            pl.BlockSpec(memory_space=pltpu.SMEM),
        ],
        scratch_shapes=[pltpu.VMEM((A_TOT, 1), jnp.float32)],
    )(probs)


def _dispatch_body(xf_hbm, pos_hbm, xs_hbm, idx_v, rows_v, sem_r0, sem_r1,
                   sem_s):
    wid = lax.axis_index("s") * NC + lax.axis_index("c")
    base = wid * APW
    srow = lax.rem(base, T_TOK)
    hc = APW // 2
    cr0 = pltpu.async_copy(xf_hbm.at[pl.ds(srow, hc), :],
                           rows_v.at[pl.ds(0, hc), :], sem_r0)
    cr1 = pltpu.async_copy(xf_hbm.at[pl.ds(srow + hc, hc), :],
                           rows_v.at[pl.ds(hc, hc), :], sem_r1)
    pltpu.sync_copy(pos_hbm.at[pl.ds(base, hc)], idx_v.at[0])
    pltpu.sync_copy(pos_hbm.at[pl.ds(base + hc, hc)], idx_v.at[1])
    cr0.wait()
    cs0 = pltpu.async_copy(rows_v.at[pl.ds(0, hc), :], xs_hbm.at[idx_v.at[0]],
                           sem_s)
    cr1.wait()
    cs1 = pltpu.async_copy(rows_v.at[pl.ds(hc, hc), :],
                           xs_hbm.at[idx_v.at[1]], sem_s)
    cs0.wait()
    cs1.wait()


def _combine_body(eo_hbm, pos_hbm, wexp_hbm, out_hbm, idx0_v, idx1_v, buf0,
                  buf1, w0_v, w1_v, sem_a0, sem_b0, sem_a1, sem_b1, sem_o):
    wid = lax.axis_index("s") * NC + lax.axis_index("c")
    tb = wid * TPW
    hc = TPW // 2
    pltpu.sync_copy(pos_hbm.at[pl.ds(tb, TPW)], idx0_v)
    pltpu.sync_copy(pos_hbm.at[pl.ds(T_TOK + tb, TPW)], idx1_v)
    g0a = pltpu.async_copy(eo_hbm.at[idx0_v.at[pl.ds(0, hc)]],
                           buf0.at[pl.ds(0, hc), :], sem_a0)
    g0b = pltpu.async_copy(eo_hbm.at[idx1_v.at[pl.ds(0, hc)]],
                           buf1.at[pl.ds(0, hc), :], sem_b0)
    g1a = pltpu.async_copy(eo_hbm.at[idx0_v.at[pl.ds(hc, hc)]],
                           buf0.at[pl.ds(hc, hc), :], sem_a1)
    g1b = pltpu.async_copy(eo_hbm.at[idx1_v.at[pl.ds(hc, hc)]],
                           buf1.at[pl.ds(hc, hc), :], sem_b1)
    pltpu.sync_copy(wexp_hbm.at[pl.ds(tb, TPW), :], w0_v)
    pltpu.sync_copy(wexp_hbm.at[pl.ds(T_TOK + tb, TPW), :], w1_v)

    def body(j, carry):
        w0 = w0_v[j, :]
        w1 = w1_v[j, :]
        for v in range(HIDDEN // LANES):
            sl = pl.ds(v * LANES, LANES)
            buf0[j, sl] = buf0[j, sl] * w0 + buf1[j, sl] * w1
        return carry

    g0a.wait()
    g0b.wait()
    lax.fori_loop(0, hc, body, 0)
    co0 = pltpu.async_copy(buf0.at[pl.ds(0, hc), :],
                           out_hbm.at[pl.ds(tb, hc), :], sem_o)
    g1a.wait()
    g1b.wait()
    lax.fori_loop(hc, TPW, body, 0)
    co1 = pltpu.async_copy(buf0.at[pl.ds(hc, hc), :],
                           out_hbm.at[pl.ds(tb + hc, hc), :], sem_o)
    co0.wait()
    co1.wait()


def _ffn_kernel(te_ref, tv_ref, xs_ref, w1_ref, b1_ref, w2_ref, b2_ref,
                out_ref, w1b, w2b):
    i = pl.program_id(0)
    switched = jnp.logical_or(
        i == 0, te_ref[i] != te_ref[jnp.maximum(i - 1, 0)])

    @pl.when(switched)
    def _():
        w1b[...] = w1_ref[0].astype(jnp.bfloat16)
        w2b[...] = w2_ref[0].astype(jnp.bfloat16)

    @pl.when(tv_ref[i] > 0)
    def _():
        x_t = xs_ref[...].astype(jnp.bfloat16)
        h = jnp.dot(x_t, w1b[...], preferred_element_type=jnp.float32)
        h = jnp.maximum(h + b1_ref[0, 0, :][None, :], 0.0)
        o = jnp.dot(h.astype(jnp.bfloat16), w2b[...],
                    preferred_element_type=jnp.float32)
        out_ref[...] = o + b2_ref[0, 0, :][None, :]


def _grouped_ffn(xs, w1, b1, w2, b2, tile_expert, tile_valid):
    grid_spec = pltpu.PrefetchScalarGridSpec(
        num_scalar_prefetch=2,
        grid=(NTILES,),
        in_specs=[
            pl.BlockSpec((BM, HIDDEN), lambda i, te, tv: (i, 0)),
            pl.BlockSpec((1, HIDDEN, D_FF), lambda i, te, tv: (te[i], 0, 0)),
            pl.BlockSpec((1, 1, D_FF), lambda i, te, tv: (te[i], 0, 0)),
            pl.BlockSpec((1, D_FF, HIDDEN), lambda i, te, tv: (te[i], 0, 0)),
            pl.BlockSpec((1, 1, HIDDEN), lambda i, te, tv: (te[i], 0, 0)),
        ],
        out_specs=pl.BlockSpec((BM, HIDDEN), lambda i, te, tv: (i, 0)),
        scratch_shapes=[
            pltpu.VMEM((HIDDEN, D_FF), jnp.bfloat16),
            pltpu.VMEM((D_FF, HIDDEN), jnp.bfloat16),
        ],
    )
    return pl.pallas_call(
        _ffn_kernel,
        grid_spec=grid_spec,
        out_shape=jax.ShapeDtypeStruct((P_PAD, HIDDEN), jnp.float32),
    )(tile_expert, tile_valid, xs, w1, b1[:, None, :], w2, b2[:, None, :])


@jax.jit
def kernel(x, router_w, router_b, w1, b1, w2, b2):
    B, S, H = x.shape
    T = B * S
    xf = x.reshape(T, H)

    logits = xf @ router_w + router_b
    probs = jax.nn.softmax(logits, axis=-1)

    pos, wexp, te, tv = _meta(probs)
    posf = pos.reshape(A_TOT)

    mesh = plsc.VectorSubcoreMesh(core_axis_name="c", subcore_axis_name="s")
    dispatch = pl.kernel(
        _dispatch_body,
        mesh=mesh,
        out_type=jax.ShapeDtypeStruct((P_PAD, HIDDEN), jnp.float32),
        scratch_types=[
            pltpu.VMEM((2, APW // 2), jnp.int32),
            pltpu.VMEM((APW, HIDDEN), jnp.float32),
            pltpu.SemaphoreType.DMA,
            pltpu.SemaphoreType.DMA,
            pltpu.SemaphoreType.DMA,
        ],
    )
    combine = pl.kernel(
        _combine_body,
        mesh=mesh,
        out_type=jax.ShapeDtypeStruct((T_TOK, HIDDEN), jnp.float32),
        scratch_types=[
            pltpu.VMEM((TPW,), jnp.int32),
            pltpu.VMEM((TPW,), jnp.int32),
            pltpu.VMEM((TPW, HIDDEN), jnp.float32),
            pltpu.VMEM((TPW, HIDDEN), jnp.float32),
            pltpu.VMEM((TPW, LANES), jnp.float32),
            pltpu.VMEM((TPW, LANES), jnp.float32),
            pltpu.SemaphoreType.DMA,
            pltpu.SemaphoreType.DMA,
            pltpu.SemaphoreType.DMA,
            pltpu.SemaphoreType.DMA,
            pltpu.SemaphoreType.DMA,
        ],
    )

    xs = dispatch(xf, posf)
    eo = _grouped_ffn(xs, w1, b1, w2, b2, te, tv)
    out = combine(eo, posf, wexp)
    return out.reshape(B, S, H)
